# baseline (speedup 1.0000x reference)
import jax
import jax.numpy as jnp
import numpy as np
from jax.experimental import pallas as pl
from jax.experimental.pallas import tpu as pltpu

N = 100
M = 28
H, W = 200, 320
STUFF = 53
THING = 80
EPS_THRESH = 1000.0 * float(np.finfo(np.float32).eps)


def _body(classes_ref, boxes_ref, ml_ref, thing_ref, stuff_ref, loss_ref,
          canvas, stuff0):
    i = pl.program_id(0)

    @pl.when(i == 0)
    def _init():
        canvas[:] = jnp.zeros_like(canvas)

    @pl.when(i < N)
    def _instance():
        x1 = boxes_ref[i, 0]
        y1 = boxes_ref[i, 1]
        x2 = boxes_ref[i, 2]
        y2 = boxes_ref[i, 3]
        hf = (y2 - y1 + 1).astype(jnp.float32)
        wf = (x2 - x1 + 1).astype(jnp.float32)

        def weights(out_len, lo, size, transpose):
            if transpose:
                shape, out_dim, red_axis = (out_len, M), 0, 1
            else:
                shape, out_dim, red_axis = (M, out_len), 1, 0
            out_c = jax.lax.broadcasted_iota(
                jnp.int32, shape, out_dim).astype(jnp.float32)
            taps = jax.lax.broadcasted_iota(
                jnp.int32, shape, 1 - out_dim).astype(jnp.float32)
            inv_scale = jnp.float32(M) / size
            kernel_scale = jnp.maximum(inv_scale, 1.0)
            rel = out_c - lo.astype(jnp.float32)
            sample_f = (rel + 0.5) * inv_scale - 0.5
            x = jnp.abs(sample_f - taps) / kernel_scale
            w = jnp.maximum(0.0, 1.0 - x)
            total = jnp.sum(w, axis=red_axis, keepdims=True)
            w = jnp.where(jnp.abs(total) > EPS_THRESH,
                          w / jnp.where(total != 0.0, total, 1.0), 0.0)
            w = jnp.where((sample_f >= -0.5) & (sample_f <= M - 0.5), w, 0.0)
            return jnp.where((rel >= 0.0) & (rel < size), w, 0.0)

        w_row_t = weights(H, y1, hf, transpose=True)
        w_col = weights(W, x1, wf, transpose=False)
        ml = ml_ref[0, 0]
        t1 = jnp.dot(ml, w_col, preferred_element_type=jnp.float32,
                     precision=jax.lax.Precision.HIGHEST)
        maskv = jnp.dot(w_row_t, t1, preferred_element_type=jnp.float32,
                        precision=jax.lax.Precision.HIGHEST)
        ys = jax.lax.broadcasted_iota(jnp.int32, (H, W), 0)
        xs = jax.lax.broadcasted_iota(jnp.int32, (H, W), 1)
        inbox = (ys >= y1) & (ys <= y2) & (xs >= x1) & (xs <= x2)
        crop = thing_ref[0]
        canvas[:] += jnp.where(inbox, jnp.exp(maskv + crop) - 1.0, 0.0)

    @pl.when(i >= N)
    def _stuff():
        c = stuff_ref[0]

        @pl.when(i == N)
        def _save0():
            stuff0[:] = c

        canvas[:] += jnp.exp(c)

    @pl.when(i == N + STUFF - 1)
    def _finish():
        total = canvas[:] + jnp.float32(N)
        loss_ref[0, 0] = (jnp.sum(jnp.log(total) - stuff0[:])
                          / jnp.float32(H * W))


def kernel(mask_logits, sem_seg_logits, gt_classes, gt_boxes, gt_panoptics):
    stuff = sem_seg_logits[0, :STUFF]
    thing = sem_seg_logits[0, STUFF:]
    classes = gt_classes.astype(jnp.int32)
    boxes = gt_boxes.astype(jnp.int32)

    def ml_idx(i, cls_ref, box_ref):
        j = jnp.minimum(i, N - 1)
        return (j, cls_ref[j], 0, 0)

    def thing_idx(i, cls_ref, box_ref):
        return (cls_ref[jnp.minimum(i, N - 1)], 0, 0)

    def stuff_idx(i, cls_ref, box_ref):
        return (jnp.clip(i - N, 0, STUFF - 1), 0, 0)

    grid_spec = pltpu.PrefetchScalarGridSpec(
        num_scalar_prefetch=2,
        grid=(N + STUFF,),
        in_specs=[
            pl.BlockSpec((1, 1, M, M), ml_idx),
            pl.BlockSpec((1, H, W), thing_idx),
            pl.BlockSpec((1, H, W), stuff_idx),
        ],
        out_specs=pl.BlockSpec(memory_space=pltpu.SMEM),
        scratch_shapes=[
            pltpu.VMEM((H, W), jnp.float32),
            pltpu.VMEM((H, W), jnp.float32),
        ],
    )
    loss = pl.pallas_call(
        _body,
        grid_spec=grid_spec,
        out_shape=jax.ShapeDtypeStruct((1, 1), jnp.float32),
    )(classes, boxes, mask_logits, thing, stuff)
    return loss[0, 0]


# speedup vs baseline: 1.0945x; 1.0945x over previous
import jax
import jax.numpy as jnp
import numpy as np
from jax.experimental import pallas as pl
from jax.experimental.pallas import tpu as pltpu

N = 100
M = 28
H, W = 200, 320
STUFF = 53
THING = 80
EPS_THRESH = 1000.0 * float(np.finfo(np.float32).eps)
WIN = 128
SCH = 8
SPAD = 56
SSTEPS = SPAD // SCH
GRID = N + SSTEPS + 1


def _body(classes_ref, boxes_ref, ml_ref, thing_ref, stuff_ref, loss_ref,
          canvas, stuff0):
    i = pl.program_id(0)

    @pl.when(i == 0)
    def _init():
        canvas[:] = jnp.zeros_like(canvas)

    @pl.when(i < N)
    def _instance():
        x1 = boxes_ref[i, 0]
        y1 = boxes_ref[i, 1]
        x2 = boxes_ref[i, 2]
        y2 = boxes_ref[i, 3]
        hf = (y2 - y1 + 1).astype(jnp.float32)
        wf = (x2 - x1 + 1).astype(jnp.float32)
        ys0 = pl.multiple_of(
            jnp.minimum(y1 - jnp.remainder(y1, 8), H - WIN), 8)

        def weights(shape, out_dim, lo, size, base):
            out_c = (jax.lax.broadcasted_iota(jnp.int32, shape, out_dim)
                     + base).astype(jnp.float32)
            taps = jax.lax.broadcasted_iota(
                jnp.int32, shape, 1 - out_dim).astype(jnp.float32)
            inv_scale = jnp.float32(M) / size
            kernel_scale = jnp.maximum(inv_scale, 1.0)
            rel = out_c - lo.astype(jnp.float32)
            sample_f = (rel + 0.5) * inv_scale - 0.5
            x = jnp.abs(sample_f - taps) / kernel_scale
            w = jnp.maximum(0.0, 1.0 - x)
            total = jnp.sum(w, axis=1 - out_dim, keepdims=True)
            w = jnp.where(jnp.abs(total) > EPS_THRESH,
                          w / jnp.where(total != 0.0, total, 1.0), 0.0)
            w = jnp.where((sample_f >= -0.5) & (sample_f <= M - 0.5), w, 0.0)
            return jnp.where((rel >= 0.0) & (rel < size), w, 0.0)

        w_row_t = weights((WIN, M), 0, y1, hf, ys0)
        w_col = weights((M, W), 1, x1, wf, 0)
        ml = ml_ref[0, 0]
        t0 = jnp.dot(w_row_t, ml, preferred_element_type=jnp.float32,
                     precision=jax.lax.Precision.HIGHEST)
        maskv = jnp.dot(t0, w_col, preferred_element_type=jnp.float32,
                        precision=jax.lax.Precision.HIGHEST)
        ys = jax.lax.broadcasted_iota(jnp.int32, (WIN, W), 0) + ys0
        xs = jax.lax.broadcasted_iota(jnp.int32, (WIN, W), 1)
        inbox = (ys >= y1) & (ys <= y2) & (xs >= x1) & (xs <= x2)
        crop = thing_ref[0, pl.ds(ys0, WIN), :]
        canvas[pl.ds(ys0, WIN), :] += jnp.where(
            inbox, jnp.exp(maskv + crop) - 1.0, 0.0)

    @pl.when((i >= N) & (i < N + SSTEPS))
    def _stuff():
        c = stuff_ref[:]

        @pl.when(i == N)
        def _save0():
            stuff0[:] = c[0]

        canvas[:] += jnp.sum(jnp.exp(c), axis=0)

    @pl.when(i == GRID - 1)
    def _finish():
        total = canvas[:] + jnp.float32(N)
        loss_ref[0, 0] = (jnp.sum(jnp.log(total) - stuff0[:])
                          / jnp.float32(H * W))


def kernel(mask_logits, sem_seg_logits, gt_classes, gt_boxes, gt_panoptics):
    stuff = sem_seg_logits[0, :STUFF]
    stuff = jnp.pad(stuff, ((0, SPAD - STUFF), (0, 0), (0, 0)),
                    constant_values=-1e30)
    thing = sem_seg_logits[0, STUFF:]
    classes = gt_classes.astype(jnp.int32)
    boxes = gt_boxes.astype(jnp.int32)

    def ml_idx(i, cls_ref, box_ref):
        j = jnp.minimum(i, N - 1)
        return (j, cls_ref[j], 0, 0)

    def thing_idx(i, cls_ref, box_ref):
        return (cls_ref[jnp.minimum(i, N - 1)], 0, 0)

    def stuff_idx(i, cls_ref, box_ref):
        return (jnp.clip(i - N, 0, SSTEPS - 1), 0, 0)

    grid_spec = pltpu.PrefetchScalarGridSpec(
        num_scalar_prefetch=2,
        grid=(GRID,),
        in_specs=[
            pl.BlockSpec((1, 1, M, M), ml_idx),
            pl.BlockSpec((1, H, W), thing_idx),
            pl.BlockSpec((SCH, H, W), stuff_idx),
        ],
        out_specs=pl.BlockSpec(memory_space=pltpu.SMEM),
        scratch_shapes=[
            pltpu.VMEM((H, W), jnp.float32),
            pltpu.VMEM((H, W), jnp.float32),
        ],
    )
    loss = pl.pallas_call(
        _body,
        grid_spec=grid_spec,
        out_shape=jax.ShapeDtypeStruct((1, 1), jnp.float32),
    )(classes, boxes, mask_logits, thing, stuff)
    return loss[0, 0]


# speedup vs baseline: 1.1408x; 1.0423x over previous
import jax
import jax.numpy as jnp
import numpy as np
from jax.experimental import pallas as pl
from jax.experimental.pallas import tpu as pltpu

N = 100
M = 28
H, W = 200, 320
STUFF = 53
THING = 80
EPS_THRESH = 1000.0 * float(np.finfo(np.float32).eps)
WIN = 128
GRID = N + 1


def _body(classes_ref, boxes_ref, ml_ref, thing_ref, stuff_ref, loss_ref,
          canvas, stuff0):
    i = pl.program_id(0)

    @pl.when(i == 0)
    def _init():
        canvas[:] = jnp.zeros_like(canvas)

    @pl.when(i < N)
    def _instance():
        x1 = boxes_ref[i, 0]
        y1 = boxes_ref[i, 1]
        x2 = boxes_ref[i, 2]
        y2 = boxes_ref[i, 3]
        hf = (y2 - y1 + 1).astype(jnp.float32)
        wf = (x2 - x1 + 1).astype(jnp.float32)
        ys0 = pl.multiple_of(
            jnp.minimum(y1 - jnp.remainder(y1, 8), H - WIN), 8)

        def raw_weights(shape, out_dim, lo, size, base):
            out_c = (jax.lax.broadcasted_iota(jnp.int32, shape, out_dim)
                     + base).astype(jnp.float32)
            taps = jax.lax.broadcasted_iota(
                jnp.int32, shape, 1 - out_dim).astype(jnp.float32)
            inv_scale = jnp.float32(M) / size
            kernel_scale = jnp.maximum(inv_scale, 1.0)
            rel = out_c - lo.astype(jnp.float32)
            sample_f = (rel + 0.5) * inv_scale - 0.5
            x = jnp.abs(sample_f - taps) / kernel_scale
            return jnp.maximum(0.0, 1.0 - x)

        def out_factors(shape, out_dim, lo, size, base, total):
            out_c = (jax.lax.broadcasted_iota(jnp.int32, shape, out_dim)
                     + base).astype(jnp.float32)
            inv_scale = jnp.float32(M) / size
            rel = out_c - lo.astype(jnp.float32)
            sample_f = (rel + 0.5) * inv_scale - 0.5
            ok = ((jnp.abs(total) > EPS_THRESH)
                  & (sample_f >= -0.5) & (sample_f <= M - 0.5)
                  & (rel >= 0.0) & (rel < size))
            fac = jnp.where(ok, 1.0 / jnp.where(total != 0.0, total, 1.0), 0.0)
            inb = (rel >= 0.0) & (rel <= (y2 - y1 if out_dim == 0
                                          else x2 - x1).astype(jnp.float32))
            return fac, inb

        w_row_t = raw_weights((WIN, M), 0, y1, hf, ys0)
        w_col = raw_weights((M, W), 1, x1, wf, 0)
        row_tot = jnp.sum(w_row_t, axis=1, keepdims=True)
        col_tot = jnp.sum(w_col, axis=0, keepdims=True)
        row_fac, row_inb = out_factors((WIN, 1), 0, y1, hf, ys0, row_tot)
        col_fac, col_inb = out_factors((1, W), 1, x1, wf, 0, col_tot)
        ml = ml_ref[0, 0]
        t0 = jnp.dot(w_row_t, ml, preferred_element_type=jnp.float32,
                     precision=jax.lax.Precision.HIGHEST)
        maskv = jnp.dot(t0, w_col, preferred_element_type=jnp.float32,
                        precision=jax.lax.Precision.HIGHEST)
        inbox = row_inb & col_inb
        crop = thing_ref[0, pl.ds(ys0, WIN), :]
        val = maskv * (row_fac * col_fac) + crop
        canvas[pl.ds(ys0, WIN), :] += jnp.where(
            inbox, jnp.exp(val) - 1.0, 0.0)

    @pl.when(i < STUFF)
    def _stuff():
        c = stuff_ref[0]

        @pl.when(i == 0)
        def _save0():
            stuff0[:] = c

        canvas[:] += jnp.exp(c)

    @pl.when(i == GRID - 1)
    def _finish():
        total = canvas[:] + jnp.float32(N)
        loss_ref[0, 0] = (jnp.sum(jnp.log(total) - stuff0[:])
                          / jnp.float32(H * W))


def kernel(mask_logits, sem_seg_logits, gt_classes, gt_boxes, gt_panoptics):
    stuff = sem_seg_logits[0, :STUFF]
    thing = sem_seg_logits[0, STUFF:]
    classes = gt_classes.astype(jnp.int32)
    boxes = gt_boxes.astype(jnp.int32)

    def ml_idx(i, cls_ref, box_ref):
        j = jnp.minimum(i, N - 1)
        return (j, cls_ref[j], 0, 0)

    def thing_idx(i, cls_ref, box_ref):
        return (cls_ref[jnp.minimum(i, N - 1)], 0, 0)

    def stuff_idx(i, cls_ref, box_ref):
        return (jnp.minimum(i, STUFF - 1), 0, 0)

    grid_spec = pltpu.PrefetchScalarGridSpec(
        num_scalar_prefetch=2,
        grid=(GRID,),
        in_specs=[
            pl.BlockSpec((1, 1, M, M), ml_idx),
            pl.BlockSpec((1, H, W), thing_idx),
            pl.BlockSpec((1, H, W), stuff_idx),
        ],
        out_specs=pl.BlockSpec(memory_space=pltpu.SMEM),
        scratch_shapes=[
            pltpu.VMEM((H, W), jnp.float32),
            pltpu.VMEM((H, W), jnp.float32),
        ],
    )
    loss = pl.pallas_call(
        _body,
        grid_spec=grid_spec,
        out_shape=jax.ShapeDtypeStruct((1, 1), jnp.float32),
    )(classes, boxes, mask_logits, thing, stuff)
    return loss[0, 0]


# speedup vs baseline: 1.3120x; 1.1501x over previous
import jax
import jax.numpy as jnp
import numpy as np
from jax.experimental import pallas as pl
from jax.experimental.pallas import tpu as pltpu

N = 100
M = 28
H, W = 200, 320
STUFF = 53
THING = 80
EPS_THRESH = 1000.0 * float(np.finfo(np.float32).eps)
WIN = 128
GRID = N + 1


def _body(classes_ref, boxes_ref, ml_ref, thing_ref, stuff_ref, loss_ref,
          canvas, stuff0):
    i = pl.program_id(0)

    @pl.when(i == 0)
    def _init():
        canvas[:] = jnp.zeros_like(canvas)

    @pl.when(i < N)
    def _instance():
        x1 = boxes_ref[i, 0]
        y1 = boxes_ref[i, 1]
        x2 = boxes_ref[i, 2]
        y2 = boxes_ref[i, 3]
        hf = (y2 - y1 + 1).astype(jnp.float32)
        wf = (x2 - x1 + 1).astype(jnp.float32)
        ys0 = pl.multiple_of(
            jnp.minimum(y1 - jnp.remainder(y1, 8), H - WIN), 8)

        def raw_weights(shape, out_dim, lo, size, base):
            out_c = (jax.lax.broadcasted_iota(jnp.int32, shape, out_dim)
                     + base).astype(jnp.float32)
            taps = jax.lax.broadcasted_iota(
                jnp.int32, shape, 1 - out_dim).astype(jnp.float32)
            inv_scale = jnp.float32(M) / size
            kernel_scale = jnp.maximum(inv_scale, 1.0)
            rel = out_c - lo.astype(jnp.float32)
            sample_f = (rel + 0.5) * inv_scale - 0.5
            x = jnp.abs(sample_f - taps) / kernel_scale
            return jnp.maximum(0.0, 1.0 - x)

        def out_factors(shape, out_dim, lo, size, base, total):
            out_c = (jax.lax.broadcasted_iota(jnp.int32, shape, out_dim)
                     + base).astype(jnp.float32)
            inv_scale = jnp.float32(M) / size
            rel = out_c - lo.astype(jnp.float32)
            sample_f = (rel + 0.5) * inv_scale - 0.5
            ok = ((jnp.abs(total) > EPS_THRESH)
                  & (sample_f >= -0.5) & (sample_f <= M - 0.5)
                  & (rel >= 0.0) & (rel < size))
            fac = jnp.where(ok, 1.0 / jnp.where(total != 0.0, total, 1.0), 0.0)
            inb = (rel >= 0.0) & (rel <= (y2 - y1 if out_dim == 0
                                          else x2 - x1).astype(jnp.float32))
            return fac, inb

        w_row_t = raw_weights((WIN, M), 0, y1, hf, ys0)
        w_col = raw_weights((M, W), 1, x1, wf, 0)
        row_tot = jnp.sum(w_row_t, axis=1, keepdims=True)
        col_tot = jnp.sum(w_col, axis=0, keepdims=True)
        row_fac, row_inb = out_factors((WIN, 1), 0, y1, hf, ys0, row_tot)
        col_fac, col_inb = out_factors((1, W), 1, x1, wf, 0, col_tot)
        ml = ml_ref[0, 0]
        t0 = jnp.dot(w_row_t, ml, preferred_element_type=jnp.float32,
                     precision=jax.lax.Precision.HIGHEST)
        maskv = jnp.dot(t0, w_col, preferred_element_type=jnp.float32,
                        precision=jax.lax.Precision.HIGHEST)
        inbox = row_inb & col_inb
        crop = thing_ref[0, 0, pl.ds(ys0, WIN), :]
        val = maskv * (row_fac * col_fac) + crop
        canvas[pl.ds(ys0, WIN), :] += jnp.where(
            inbox, jnp.exp(val) - 1.0, 0.0)

    @pl.when(i < STUFF)
    def _stuff():
        c = stuff_ref[0, 0]

        @pl.when(i == 0)
        def _save0():
            stuff0[:] = c

        canvas[:] += jnp.exp(c)

    @pl.when(i == GRID - 1)
    def _finish():
        total = canvas[:] + jnp.float32(N)
        loss_ref[0, 0] = (jnp.sum(jnp.log(total) - stuff0[:])
                          / jnp.float32(H * W))


def kernel(mask_logits, sem_seg_logits, gt_classes, gt_boxes, gt_panoptics):
    classes = gt_classes.astype(jnp.int32)
    boxes = gt_boxes.astype(jnp.int32)

    def ml_idx(i, cls_ref, box_ref):
        j = jnp.minimum(i, N - 1)
        return (j, cls_ref[j], 0, 0)

    def thing_idx(i, cls_ref, box_ref):
        return (0, STUFF + cls_ref[jnp.minimum(i, N - 1)], 0, 0)

    def stuff_idx(i, cls_ref, box_ref):
        return (0, jnp.minimum(i, STUFF - 1), 0, 0)

    grid_spec = pltpu.PrefetchScalarGridSpec(
        num_scalar_prefetch=2,
        grid=(GRID,),
        in_specs=[
            pl.BlockSpec((1, 1, M, M), ml_idx),
            pl.BlockSpec((1, 1, H, W), thing_idx),
            pl.BlockSpec((1, 1, H, W), stuff_idx),
        ],
        out_specs=pl.BlockSpec(memory_space=pltpu.SMEM),
        scratch_shapes=[
            pltpu.VMEM((H, W), jnp.float32),
            pltpu.VMEM((H, W), jnp.float32),
        ],
    )
    loss = pl.pallas_call(
        _body,
        grid_spec=grid_spec,
        out_shape=jax.ShapeDtypeStruct((1, 1), jnp.float32),
    )(classes, boxes, mask_logits, sem_seg_logits, sem_seg_logits)
    return loss[0, 0]


# speedup vs baseline: 3.8496x; 2.9342x over previous
import jax
import jax.numpy as jnp
import numpy as np
from jax.experimental import pallas as pl
from jax.experimental.pallas import tpu as pltpu

N = 100
M = 28
H, W = 200, 320
STUFF = 53
THING = 80
EPS_THRESH = 1000.0 * float(np.finfo(np.float32).eps)
WIN = 128
GRID = N // 2 + 1


def _body(classes_ref, boxes_ref, ml0_ref, ml1_ref, th0_ref, th1_ref,
          st0_ref, st1_ref, loss_ref, canvas, stuff0):
    i = pl.program_id(0)

    @pl.when(i == 0)
    def _init():
        canvas[:] = jnp.zeros_like(canvas)

    def instance(inst, ml_ref, thing_ref):
        x1 = boxes_ref[inst, 0]
        y1 = boxes_ref[inst, 1]
        x2 = boxes_ref[inst, 2]
        y2 = boxes_ref[inst, 3]
        hf = (y2 - y1 + 1).astype(jnp.float32)
        wf = (x2 - x1 + 1).astype(jnp.float32)
        ys0 = pl.multiple_of(
            jnp.minimum(y1 - jnp.remainder(y1, 8), H - WIN), 8)

        def raw_weights(shape, out_dim, lo, size, base):
            out_c = (jax.lax.broadcasted_iota(jnp.int32, shape, out_dim)
                     + base).astype(jnp.float32)
            taps = jax.lax.broadcasted_iota(
                jnp.int32, shape, 1 - out_dim).astype(jnp.float32)
            inv_scale = jnp.float32(M) / size
            kernel_scale = jnp.maximum(inv_scale, 1.0)
            rel = out_c - lo.astype(jnp.float32)
            sample_f = (rel + 0.5) * inv_scale - 0.5
            x = jnp.abs(sample_f - taps) / kernel_scale
            return jnp.maximum(0.0, 1.0 - x)

        def out_factors(shape, out_dim, lo, size, base, total, hi):
            out_c = (jax.lax.broadcasted_iota(jnp.int32, shape, out_dim)
                     + base).astype(jnp.float32)
            inv_scale = jnp.float32(M) / size
            rel = out_c - lo.astype(jnp.float32)
            sample_f = (rel + 0.5) * inv_scale - 0.5
            ok = ((jnp.abs(total) > EPS_THRESH)
                  & (sample_f >= -0.5) & (sample_f <= M - 0.5)
                  & (rel >= 0.0) & (rel < size))
            fac = jnp.where(ok, 1.0 / jnp.where(total != 0.0, total, 1.0), 0.0)
            inb = (rel >= 0.0) & (rel <= hi.astype(jnp.float32))
            return fac, inb

        w_row_t = raw_weights((WIN, M), 0, y1, hf, ys0)
        w_col = raw_weights((M, W), 1, x1, wf, 0)
        row_tot = jnp.sum(w_row_t, axis=1, keepdims=True)
        col_tot = jnp.sum(w_col, axis=0, keepdims=True)
        row_fac, row_inb = out_factors((WIN, 1), 0, y1, hf, ys0, row_tot,
                                       y2 - y1)
        col_fac, col_inb = out_factors((1, W), 1, x1, wf, 0, col_tot,
                                       x2 - x1)
        ml = ml_ref[0].astype(jnp.bfloat16)
        t0 = jnp.dot(w_row_t.astype(jnp.bfloat16), ml,
                     preferred_element_type=jnp.float32)
        maskv = jnp.dot(t0.astype(jnp.bfloat16), w_col.astype(jnp.bfloat16),
                        preferred_element_type=jnp.float32)
        inbox = row_inb & col_inb
        crop = thing_ref[0, 0, pl.ds(ys0, WIN), :]
        val = maskv * (row_fac * col_fac) + crop
        canvas[pl.ds(ys0, WIN), :] += jnp.where(
            inbox, jnp.exp(val) - 1.0, 0.0)

    @pl.when(i < N // 2)
    def _instances():
        instance(2 * i, ml0_ref, th0_ref)
        instance(2 * i + 1, ml1_ref, th1_ref)

    @pl.when(2 * i < STUFF)
    def _stuff():
        c0 = st0_ref[0, 0]

        @pl.when(i == 0)
        def _save0():
            stuff0[:] = c0

        acc = jnp.exp(c0)

        @pl.when(2 * i + 1 < STUFF)
        def _pair():
            canvas[:] += acc + jnp.exp(st1_ref[0, 0])

        @pl.when(2 * i + 1 >= STUFF)
        def _single():
            canvas[:] += acc

    @pl.when(i == GRID - 1)
    def _finish():
        total = canvas[:] + jnp.float32(N)
        loss_ref[0, 0] = (jnp.sum(jnp.log(total) - stuff0[:])
                          / jnp.float32(H * W))


def kernel(mask_logits, sem_seg_logits, gt_classes, gt_boxes, gt_panoptics):
    classes = gt_classes.astype(jnp.int32)
    boxes = gt_boxes.astype(jnp.int32)
    onehot = (classes[:, None] == jnp.arange(THING)[None, :]
              ).astype(jnp.float32)
    ml_sel = jnp.sum(mask_logits * onehot[:, :, None, None], axis=1)

    def ml_idx(par):
        def f(i, cls_ref, box_ref):
            return (jnp.minimum(2 * i + par, N - 1), 0, 0)
        return f

    def thing_idx(par):
        def f(i, cls_ref, box_ref):
            return (0, STUFF + cls_ref[jnp.minimum(2 * i + par, N - 1)], 0, 0)
        return f

    def stuff_idx(par):
        def f(i, cls_ref, box_ref):
            return (0, jnp.minimum(2 * i + par, STUFF - 1), 0, 0)
        return f

    grid_spec = pltpu.PrefetchScalarGridSpec(
        num_scalar_prefetch=2,
        grid=(GRID,),
        in_specs=[
            pl.BlockSpec((1, M, M), ml_idx(0)),
            pl.BlockSpec((1, M, M), ml_idx(1)),
            pl.BlockSpec((1, 1, H, W), thing_idx(0)),
            pl.BlockSpec((1, 1, H, W), thing_idx(1)),
            pl.BlockSpec((1, 1, H, W), stuff_idx(0)),
            pl.BlockSpec((1, 1, H, W), stuff_idx(1)),
        ],
        out_specs=pl.BlockSpec(memory_space=pltpu.SMEM),
        scratch_shapes=[
            pltpu.VMEM((H, W), jnp.float32),
            pltpu.VMEM((H, W), jnp.float32),
        ],
    )
    loss = pl.pallas_call(
        _body,
        grid_spec=grid_spec,
        out_shape=jax.ShapeDtypeStruct((1, 1), jnp.float32),
    )(classes, boxes, ml_sel, ml_sel, sem_seg_logits, sem_seg_logits,
      sem_seg_logits, sem_seg_logits)
    return loss[0, 0]


# speedup vs baseline: 4.8143x; 1.2506x over previous
import jax
import jax.numpy as jnp
import numpy as np
from jax.experimental import pallas as pl
from jax.experimental.pallas import tpu as pltpu

N = 100
M = 28
H, W = 200, 320
STUFF = 53
THING = 80
EPS_THRESH = 1000.0 * float(np.finfo(np.float32).eps)
WIN = 128
P = 4
Q = 3
GRID = N // P + 1


def _body(classes_ref, boxes_ref, *refs):
    ml_refs = refs[:P]
    th_refs = refs[P:2 * P]
    st_refs = refs[2 * P:2 * P + Q]
    loss_ref = refs[2 * P + Q]
    canvas, stuff0 = refs[2 * P + Q + 1:]
    i = pl.program_id(0)

    @pl.when(i == 0)
    def _init():
        canvas[:] = jnp.zeros_like(canvas)

    def instance(inst, ml_ref, thing_ref):
        x1 = boxes_ref[inst, 0]
        y1 = boxes_ref[inst, 1]
        x2 = boxes_ref[inst, 2]
        y2 = boxes_ref[inst, 3]
        hf = (y2 - y1 + 1).astype(jnp.float32)
        wf = (x2 - x1 + 1).astype(jnp.float32)
        ys0 = pl.multiple_of(
            jnp.minimum(y1 - jnp.remainder(y1, 8), H - WIN), 8)

        def raw_weights(shape, out_dim, lo, size, base):
            out_c = (jax.lax.broadcasted_iota(jnp.int32, shape, out_dim)
                     + base).astype(jnp.float32)
            taps = jax.lax.broadcasted_iota(
                jnp.int32, shape, 1 - out_dim).astype(jnp.float32)
            inv_scale = jnp.float32(M) / size
            kernel_scale = jnp.maximum(inv_scale, 1.0)
            rel = out_c - lo.astype(jnp.float32)
            sample_f = (rel + 0.5) * inv_scale - 0.5
            x = jnp.abs(sample_f - taps) / kernel_scale
            return jnp.maximum(0.0, 1.0 - x)

        def out_factors(shape, out_dim, lo, size, base, total, hi):
            out_c = (jax.lax.broadcasted_iota(jnp.int32, shape, out_dim)
                     + base).astype(jnp.float32)
            inv_scale = jnp.float32(M) / size
            rel = out_c - lo.astype(jnp.float32)
            sample_f = (rel + 0.5) * inv_scale - 0.5
            ok = ((jnp.abs(total) > EPS_THRESH)
                  & (sample_f >= -0.5) & (sample_f <= M - 0.5)
                  & (rel >= 0.0) & (rel < size))
            fac = jnp.where(ok, 1.0 / jnp.where(total != 0.0, total, 1.0), 0.0)
            inb = (rel >= 0.0) & (rel <= hi.astype(jnp.float32))
            return fac, inb

        w_row_t = raw_weights((WIN, M), 0, y1, hf, ys0)
        w_col = raw_weights((M, W), 1, x1, wf, 0)
        row_tot = jnp.sum(w_row_t, axis=1, keepdims=True)
        col_tot = jnp.sum(w_col, axis=0, keepdims=True)
        row_fac, row_inb = out_factors((WIN, 1), 0, y1, hf, ys0, row_tot,
                                       y2 - y1)
        col_fac, col_inb = out_factors((1, W), 1, x1, wf, 0, col_tot,
                                       x2 - x1)
        ml = ml_ref[0].astype(jnp.bfloat16)
        t0 = jnp.dot(w_row_t.astype(jnp.bfloat16), ml,
                     preferred_element_type=jnp.float32)
        maskv = jnp.dot(t0.astype(jnp.bfloat16), w_col.astype(jnp.bfloat16),
                        preferred_element_type=jnp.float32)
        inbox = row_inb & col_inb
        crop = thing_ref[0, 0, pl.ds(ys0, WIN), :]
        val = maskv * (row_fac * col_fac) + crop
        canvas[pl.ds(ys0, WIN), :] += jnp.where(
            inbox, jnp.exp(val) - 1.0, 0.0)

    @pl.when(i < N // P)
    def _instances():
        for p in range(P):
            instance(P * i + p, ml_refs[p], th_refs[p])

    @pl.when(Q * i < STUFF)
    def _stuff():
        c0 = st_refs[0][0, 0]

        @pl.when(i == 0)
        def _save0():
            stuff0[:] = c0

        acc = jnp.exp(c0)
        for q in range(1, Q):
            valid = (Q * i + q < STUFF).astype(jnp.float32)
            acc = acc + jnp.exp(st_refs[q][0, 0]) * valid
        canvas[:] += acc

    @pl.when(i == GRID - 1)
    def _finish():
        total = canvas[:] + jnp.float32(N)
        loss_ref[0, 0] = (jnp.sum(jnp.log(total) - stuff0[:])
                          / jnp.float32(H * W))


def kernel(mask_logits, sem_seg_logits, gt_classes, gt_boxes, gt_panoptics):
    classes = gt_classes.astype(jnp.int32)
    boxes = gt_boxes.astype(jnp.int32)
    onehot = (classes[:, None] == jnp.arange(THING)[None, :]
              ).astype(jnp.float32)
    ml_sel = jnp.sum(mask_logits * onehot[:, :, None, None], axis=1)

    def ml_idx(par):
        def f(i, cls_ref, box_ref):
            return (jnp.minimum(P * i + par, N - 1), 0, 0)
        return f

    def thing_idx(par):
        def f(i, cls_ref, box_ref):
            return (0, STUFF + cls_ref[jnp.minimum(P * i + par, N - 1)], 0, 0)
        return f

    def stuff_idx(par):
        def f(i, cls_ref, box_ref):
            return (0, jnp.minimum(Q * i + par, STUFF - 1), 0, 0)
        return f

    grid_spec = pltpu.PrefetchScalarGridSpec(
        num_scalar_prefetch=2,
        grid=(GRID,),
        in_specs=(
            [pl.BlockSpec((1, M, M), ml_idx(p)) for p in range(P)]
            + [pl.BlockSpec((1, 1, H, W), thing_idx(p)) for p in range(P)]
            + [pl.BlockSpec((1, 1, H, W), stuff_idx(q)) for q in range(Q)]
        ),
        out_specs=pl.BlockSpec(memory_space=pltpu.SMEM),
        scratch_shapes=[
            pltpu.VMEM((H, W), jnp.float32),
            pltpu.VMEM((H, W), jnp.float32),
        ],
    )
    loss = pl.pallas_call(
        _body,
        grid_spec=grid_spec,
        out_shape=jax.ShapeDtypeStruct((1, 1), jnp.float32),
    )(classes, boxes, *([ml_sel] * P), *([sem_seg_logits] * P),
      *([sem_seg_logits] * Q))
    return loss[0, 0]


# speedup vs baseline: 4.9418x; 1.0265x over previous
import jax
import jax.numpy as jnp
import numpy as np
from jax.experimental import pallas as pl
from jax.experimental.pallas import tpu as pltpu

N = 100
M = 28
H, W = 200, 320
STUFF = 53
THING = 80
EPS_THRESH = 1000.0 * float(np.finfo(np.float32).eps)
WIN = 128
P = 5
Q = 3
GRID = N // P + 1


def _body(classes_ref, boxes_ref, *refs):
    ml_refs = refs[:P]
    th_refs = refs[P:2 * P]
    st_refs = refs[2 * P:2 * P + Q]
    loss_ref = refs[2 * P + Q]
    canvas, stuff0 = refs[2 * P + Q + 1:]
    i = pl.program_id(0)

    @pl.when(i == 0)
    def _init():
        canvas[:] = jnp.zeros_like(canvas)

    def instance(inst, ml_ref, thing_ref):
        x1 = boxes_ref[inst, 0]
        y1 = boxes_ref[inst, 1]
        x2 = boxes_ref[inst, 2]
        y2 = boxes_ref[inst, 3]
        hf = (y2 - y1 + 1).astype(jnp.float32)
        wf = (x2 - x1 + 1).astype(jnp.float32)
        ys0 = pl.multiple_of(
            jnp.minimum(y1 - jnp.remainder(y1, 8), H - WIN), 8)

        def raw_weights(shape, out_dim, lo, size, base):
            out_c = (jax.lax.broadcasted_iota(jnp.int32, shape, out_dim)
                     + base).astype(jnp.float32)
            taps = jax.lax.broadcasted_iota(
                jnp.int32, shape, 1 - out_dim).astype(jnp.float32)
            inv_scale = jnp.float32(M) / size
            kernel_scale = jnp.maximum(inv_scale, 1.0)
            rel = out_c - lo.astype(jnp.float32)
            sample_f = (rel + 0.5) * inv_scale - 0.5
            x = jnp.abs(sample_f - taps) / kernel_scale
            return jnp.maximum(0.0, 1.0 - x)

        def out_factors(shape, out_dim, lo, size, base, total, hi):
            out_c = (jax.lax.broadcasted_iota(jnp.int32, shape, out_dim)
                     + base).astype(jnp.float32)
            inv_scale = jnp.float32(M) / size
            rel = out_c - lo.astype(jnp.float32)
            sample_f = (rel + 0.5) * inv_scale - 0.5
            ok = ((jnp.abs(total) > EPS_THRESH)
                  & (sample_f >= -0.5) & (sample_f <= M - 0.5)
                  & (rel >= 0.0) & (rel < size))
            fac = jnp.where(ok, 1.0 / jnp.where(total != 0.0, total, 1.0), 0.0)
            inb = (rel >= 0.0) & (rel <= hi.astype(jnp.float32))
            return fac, inb

        w_row_t = raw_weights((WIN, M), 0, y1, hf, ys0)
        w_col = raw_weights((M, W), 1, x1, wf, 0)
        row_tot = jnp.sum(w_row_t, axis=1, keepdims=True)
        col_tot = jnp.sum(w_col, axis=0, keepdims=True)
        row_fac, row_inb = out_factors((WIN, 1), 0, y1, hf, ys0, row_tot,
                                       y2 - y1)
        col_fac, col_inb = out_factors((1, W), 1, x1, wf, 0, col_tot,
                                       x2 - x1)
        ml = ml_ref[0].astype(jnp.bfloat16)
        t0 = jnp.dot(w_row_t.astype(jnp.bfloat16), ml,
                     preferred_element_type=jnp.float32)
        maskv = jnp.dot(t0.astype(jnp.bfloat16), w_col.astype(jnp.bfloat16),
                        preferred_element_type=jnp.float32)
        inbox = row_inb & col_inb
        crop = thing_ref[0, 0, pl.ds(ys0, WIN), :]
        val = maskv * (row_fac * col_fac) + crop
        canvas[pl.ds(ys0, WIN), :] += jnp.where(
            inbox, jnp.exp(val) - 1.0, 0.0)

    @pl.when(i < N // P)
    def _instances():
        for p in range(P):
            instance(P * i + p, ml_refs[p], th_refs[p])

    @pl.when(Q * i < STUFF)
    def _stuff():
        c0 = st_refs[0][0, 0]

        @pl.when(i == 0)
        def _save0():
            stuff0[:] = c0

        acc = jnp.exp(c0)
        for q in range(1, Q):
            valid = (Q * i + q < STUFF).astype(jnp.float32)
            acc = acc + jnp.exp(st_refs[q][0, 0]) * valid
        canvas[:] += acc

    @pl.when(i == GRID - 1)
    def _finish():
        total = canvas[:] + jnp.float32(N)
        loss_ref[0, 0] = (jnp.sum(jnp.log(total) - stuff0[:])
                          / jnp.float32(H * W))


def kernel(mask_logits, sem_seg_logits, gt_classes, gt_boxes, gt_panoptics):
    classes = gt_classes.astype(jnp.int32)
    boxes = gt_boxes.astype(jnp.int32)
    onehot = (classes[:, None] == jnp.arange(THING)[None, :]
              ).astype(jnp.float32)
    ml_sel = jnp.sum(mask_logits * onehot[:, :, None, None], axis=1)

    def ml_idx(par):
        def f(i, cls_ref, box_ref):
            return (jnp.minimum(P * i + par, N - 1), 0, 0)
        return f

    def thing_idx(par):
        def f(i, cls_ref, box_ref):
            return (0, STUFF + cls_ref[jnp.minimum(P * i + par, N - 1)], 0, 0)
        return f

    def stuff_idx(par):
        def f(i, cls_ref, box_ref):
            return (0, jnp.minimum(Q * i + par, STUFF - 1), 0, 0)
        return f

    grid_spec = pltpu.PrefetchScalarGridSpec(
        num_scalar_prefetch=2,
        grid=(GRID,),
        in_specs=(
            [pl.BlockSpec((1, M, M), ml_idx(p)) for p in range(P)]
            + [pl.BlockSpec((1, 1, H, W), thing_idx(p)) for p in range(P)]
            + [pl.BlockSpec((1, 1, H, W), stuff_idx(q)) for q in range(Q)]
        ),
        out_specs=pl.BlockSpec(memory_space=pltpu.SMEM),
        scratch_shapes=[
            pltpu.VMEM((H, W), jnp.float32),
            pltpu.VMEM((H, W), jnp.float32),
        ],
    )
    loss = pl.pallas_call(
        _body,
        grid_spec=grid_spec,
        out_shape=jax.ShapeDtypeStruct((1, 1), jnp.float32),
    )(classes, boxes, *([ml_sel] * P), *([sem_seg_logits] * P),
      *([sem_seg_logits] * Q))
    return loss[0, 0]


# speedup vs baseline: 5.0771x; 1.0274x over previous
import jax
import jax.numpy as jnp
import numpy as np
from jax.experimental import pallas as pl
from jax.experimental.pallas import tpu as pltpu

N = 100
M = 28
H, W = 200, 320
STUFF = 53
THING = 80
EPS_THRESH = 1000.0 * float(np.finfo(np.float32).eps)
WIN = 128
P = 10
Q = 6
GRID = N // P + 1


def _body(classes_ref, boxes_ref, *refs):
    ml_refs = refs[:P]
    th_refs = refs[P:2 * P]
    st_refs = refs[2 * P:2 * P + Q]
    loss_ref = refs[2 * P + Q]
    canvas, stuff0 = refs[2 * P + Q + 1:]
    i = pl.program_id(0)

    @pl.when(i == 0)
    def _init():
        canvas[:] = jnp.zeros_like(canvas)

    def instance(inst, ml_ref, thing_ref):
        x1 = boxes_ref[inst, 0]
        y1 = boxes_ref[inst, 1]
        x2 = boxes_ref[inst, 2]
        y2 = boxes_ref[inst, 3]
        hf = (y2 - y1 + 1).astype(jnp.float32)
        wf = (x2 - x1 + 1).astype(jnp.float32)
        ys0 = pl.multiple_of(
            jnp.minimum(y1 - jnp.remainder(y1, 8), H - WIN), 8)

        def raw_weights(shape, out_dim, lo, size, base):
            out_c = (jax.lax.broadcasted_iota(jnp.int32, shape, out_dim)
                     + base).astype(jnp.float32)
            taps = jax.lax.broadcasted_iota(
                jnp.int32, shape, 1 - out_dim).astype(jnp.float32)
            inv_scale = jnp.float32(M) / size
            kernel_scale = jnp.maximum(inv_scale, 1.0)
            rel = out_c - lo.astype(jnp.float32)
            sample_f = (rel + 0.5) * inv_scale - 0.5
            x = jnp.abs(sample_f - taps) / kernel_scale
            return jnp.maximum(0.0, 1.0 - x)

        def out_factors(shape, out_dim, lo, size, base, total, hi):
            out_c = (jax.lax.broadcasted_iota(jnp.int32, shape, out_dim)
                     + base).astype(jnp.float32)
            inv_scale = jnp.float32(M) / size
            rel = out_c - lo.astype(jnp.float32)
            sample_f = (rel + 0.5) * inv_scale - 0.5
            ok = ((jnp.abs(total) > EPS_THRESH)
                  & (sample_f >= -0.5) & (sample_f <= M - 0.5)
                  & (rel >= 0.0) & (rel < size))
            fac = jnp.where(ok, 1.0 / jnp.where(total != 0.0, total, 1.0), 0.0)
            inb = (rel >= 0.0) & (rel <= hi.astype(jnp.float32))
            return fac, inb

        w_row_t = raw_weights((WIN, M), 0, y1, hf, ys0)
        w_col = raw_weights((M, W), 1, x1, wf, 0)
        row_tot = jnp.sum(w_row_t, axis=1, keepdims=True)
        col_tot = jnp.sum(w_col, axis=0, keepdims=True)
        row_fac, row_inb = out_factors((WIN, 1), 0, y1, hf, ys0, row_tot,
                                       y2 - y1)
        col_fac, col_inb = out_factors((1, W), 1, x1, wf, 0, col_tot,
                                       x2 - x1)
        ml = ml_ref[0].astype(jnp.bfloat16)
        t0 = jnp.dot(w_row_t.astype(jnp.bfloat16), ml,
                     preferred_element_type=jnp.float32)
        maskv = jnp.dot(t0.astype(jnp.bfloat16), w_col.astype(jnp.bfloat16),
                        preferred_element_type=jnp.float32)
        inbox = row_inb & col_inb
        crop = thing_ref[0, 0, pl.ds(ys0, WIN), :]
        val = maskv * (row_fac * col_fac) + crop
        canvas[pl.ds(ys0, WIN), :] += jnp.where(
            inbox, jnp.exp(val) - 1.0, 0.0)

    @pl.when(i < N // P)
    def _instances():
        for p in range(P):
            instance(P * i + p, ml_refs[p], th_refs[p])

    @pl.when(Q * i < STUFF)
    def _stuff():
        c0 = st_refs[0][0, 0]

        @pl.when(i == 0)
        def _save0():
            stuff0[:] = c0

        acc = jnp.exp(c0)
        for q in range(1, Q):
            valid = (Q * i + q < STUFF).astype(jnp.float32)
            acc = acc + jnp.exp(st_refs[q][0, 0]) * valid
        canvas[:] += acc

    @pl.when(i == GRID - 1)
    def _finish():
        total = canvas[:] + jnp.float32(N)
        loss_ref[0, 0] = (jnp.sum(jnp.log(total) - stuff0[:])
                          / jnp.float32(H * W))


def kernel(mask_logits, sem_seg_logits, gt_classes, gt_boxes, gt_panoptics):
    classes = gt_classes.astype(jnp.int32)
    boxes = gt_boxes.astype(jnp.int32)
    onehot = (classes[:, None] == jnp.arange(THING)[None, :]
              ).astype(jnp.float32)
    ml_sel = jnp.sum(mask_logits * onehot[:, :, None, None], axis=1)

    def ml_idx(par):
        def f(i, cls_ref, box_ref):
            return (jnp.minimum(P * i + par, N - 1), 0, 0)
        return f

    def thing_idx(par):
        def f(i, cls_ref, box_ref):
            return (0, STUFF + cls_ref[jnp.minimum(P * i + par, N - 1)], 0, 0)
        return f

    def stuff_idx(par):
        def f(i, cls_ref, box_ref):
            return (0, jnp.minimum(Q * i + par, STUFF - 1), 0, 0)
        return f

    grid_spec = pltpu.PrefetchScalarGridSpec(
        num_scalar_prefetch=2,
        grid=(GRID,),
        in_specs=(
            [pl.BlockSpec((1, M, M), ml_idx(p)) for p in range(P)]
            + [pl.BlockSpec((1, 1, H, W), thing_idx(p)) for p in range(P)]
            + [pl.BlockSpec((1, 1, H, W), stuff_idx(q)) for q in range(Q)]
        ),
        out_specs=pl.BlockSpec(memory_space=pltpu.SMEM),
        scratch_shapes=[
            pltpu.VMEM((H, W), jnp.float32),
            pltpu.VMEM((H, W), jnp.float32),
        ],
    )
    loss = pl.pallas_call(
        _body,
        grid_spec=grid_spec,
        out_shape=jax.ShapeDtypeStruct((1, 1), jnp.float32),
    )(classes, boxes, *([ml_sel] * P), *([sem_seg_logits] * P),
      *([sem_seg_logits] * Q))
    return loss[0, 0]
